# baseline (speedup 1.0000x reference)
import functools

import jax
import jax.numpy as jnp
from jax import lax
from jax.experimental import pallas as pl
from jax.experimental.pallas import tpu as pltpu
from jax.experimental.pallas import tpu_sc as plsc


def _gather_padded(xT2, emb2, pos, *, B, S, D, NC, NS):
    NW = NC * NS
    WB = B // NW
    NB = 4
    W = 2 * D
    assert B % NW == 0 and WB <= 128 and S % NB == 0

    mesh = plsc.VectorSubcoreMesh(core_axis_name="c", subcore_axis_name="s")

    LANES = D // 16

    @functools.partial(
        pl.kernel,
        out_type=jax.ShapeDtypeStruct((B, S * W), jnp.float32),
        mesh=mesh,
        scratch_types=[
            pltpu.VMEM((S, WB), jnp.int32),
            pltpu.VMEM((S, D), jnp.float32),
            pltpu.VMEM((NB, WB, D), jnp.float32),
            [pltpu.SemaphoreType.DMA] * NB,
            [pltpu.SemaphoreType.DMA] * NB,
        ],
        compiler_params=pltpu.CompilerParams(use_tc_tiling_on_sc=False),
    )
    def emb_kernel(x_hbm, emb_hbm, pos_hbm, out_hbm, idx_v, pos_v, rows_v,
                   gsems, ssems):
        bc = lax.axis_index("s") * NC + lax.axis_index("c")

        pltpu.sync_copy(pos_hbm, pos_v)
        pltpu.sync_copy(x_hbm.at[:, pl.ds(bc * WB, WB)], idx_v)

        def gather(s, b):
            return pltpu.make_async_copy(
                emb_hbm.at[idx_v.at[s]], rows_v.at[b], gsems[b])

        def scatter(s, b):
            return pltpu.make_async_copy(
                rows_v.at[b],
                out_hbm.at[pl.ds(bc * WB, WB), pl.ds(s * W, D)],
                ssems[b])

        gather(0, 0).start()
        gather(1, 1).start()

        @pl.loop(0, S, step=NB)
        def _(s0):
            for b in range(NB):
                s = s0 + b
                tb = (b + 2) % NB

                @pl.when(jnp.logical_and(s + 2 < S, s >= 2))
                def _():
                    scatter(s - 2, tb).wait()

                @pl.when(s + 2 < S)
                def _():
                    gather(s + 2, tb).start()

                gather(s, b).wait()

                pc = [pos_v[s, pl.ds(16 * c, 16)] for c in range(LANES)]

                @pl.loop(0, WB, unroll=8)
                def _(i):
                    for c in range(LANES):
                        plsc.addupdate(
                            rows_v.at[b, i, pl.ds(16 * c, 16)], pc[c])

                scatter(s, b).start()

        for b in range(NB):
            scatter(S - NB + b, b).wait()

    return emb_kernel(xT2, emb2, pos)


def _pad_rows_tc(embT, *, V, D, C=512):
    def body(src_ref, eye_ref, dst_ref):
        t = src_ref[...]
        tT = lax.dot_general(
            t, eye_ref[...], (((0,), (0,)), ((), ())),
            precision=lax.Precision.HIGHEST,
            preferred_element_type=jnp.float32)
        dst_ref[...] = jnp.pad(tT, ((0, 0), (0, D)))

    grid = (V + C - 1) // C
    return pl.pallas_call(
        body,
        grid=(grid,),
        in_specs=[
            pl.BlockSpec((D, C), lambda j: (0, j)),
            pl.BlockSpec((D, D), lambda j: (0, 0)),
        ],
        out_specs=pl.BlockSpec((C, 2 * D), lambda j: (j, 0)),
        out_shape=jax.ShapeDtypeStruct((V, 2 * D), jnp.float32),
    )(embT, jnp.eye(D, dtype=jnp.float32))


def kernel(x, emb_table, pos_table):
    B, S = x.shape
    V, D = emb_table.shape
    assert pos_table.shape == (S, D)

    info = plsc.get_sparse_core_info()
    NC, NS = info.num_cores, info.num_subcores

    xT2 = x.T.astype(jnp.int32) * 2

    emb2 = _pad_rows_tc(emb_table.T, V=V, D=D).reshape(2 * V, D)

    padded = _gather_padded(xT2, emb2, pos_table, B=B, S=S, D=D, NC=NC, NS=NS)
    return padded.reshape(B, S, 2 * D)[:, :, :D]


# speedup vs baseline: 2.0092x; 2.0092x over previous
import functools

import jax
import jax.numpy as jnp
from jax import lax
from jax.experimental import pallas as pl
from jax.experimental.pallas import tpu as pltpu
from jax.experimental.pallas import tpu_sc as plsc


def _gather_padded(xT2, emb2, pos, *, B, S, D, NC, NS):
    NW = NC * NS
    WB = B // NW
    NB = 4
    W = 2 * D
    assert B % NW == 0 and WB <= 128 and S % NB == 0

    mesh = plsc.VectorSubcoreMesh(core_axis_name="c", subcore_axis_name="s")

    LANES = D // 16

    @functools.partial(
        pl.kernel,
        out_type=jax.ShapeDtypeStruct((B, S * W), jnp.float32),
        mesh=mesh,
        scratch_types=[
            pltpu.VMEM((S, WB), jnp.int32),
            pltpu.VMEM((S, D), jnp.float32),
            pltpu.VMEM((NB, WB, D), jnp.float32),
            [pltpu.SemaphoreType.DMA] * NB,
            [pltpu.SemaphoreType.DMA] * NB,
        ],
        compiler_params=pltpu.CompilerParams(use_tc_tiling_on_sc=False),
    )
    def emb_kernel(x_hbm, emb_hbm, pos_hbm, out_hbm, idx_v, pos_v, rows_v,
                   gsems, ssems):
        bc = lax.axis_index("s") * NC + lax.axis_index("c")

        pltpu.sync_copy(pos_hbm, pos_v)
        pltpu.sync_copy(x_hbm.at[:, pl.ds(bc * WB, WB)], idx_v)

        def gather(s, b):
            return pltpu.make_async_copy(
                emb_hbm.at[idx_v.at[s]], rows_v.at[b], gsems[b])

        def scatter(s, b):
            return pltpu.make_async_copy(
                rows_v.at[b],
                out_hbm.at[pl.ds(bc * WB, WB), pl.ds(s * W, D)],
                ssems[b])

        gather(0, 0).start()
        gather(1, 1).start()

        @pl.loop(0, S, step=NB)
        def _(s0):
            for b in range(NB):
                s = s0 + b
                tb = (b + 2) % NB

                @pl.when(jnp.logical_and(s + 2 < S, s >= 2))
                def _():
                    scatter(s - 2, tb).wait()

                @pl.when(s + 2 < S)
                def _():
                    gather(s + 2, tb).start()

                gather(s, b).wait()

                pc = [pos_v[s, pl.ds(16 * c, 16)] for c in range(LANES)]

                @pl.loop(0, WB, unroll=8)
                def _(i):
                    for c in range(LANES):
                        plsc.addupdate(
                            rows_v.at[b, i, pl.ds(16 * c, 16)], pc[c])

                scatter(s, b).start()

        for b in range(NB):
            scatter(S - NB + b, b).wait()

    return emb_kernel(xT2, emb2, pos)


def kernel(x, emb_table, pos_table):
    B, S = x.shape
    V, D = emb_table.shape
    assert pos_table.shape == (S, D)

    info = plsc.get_sparse_core_info()
    NC, NS = info.num_cores, info.num_subcores

    xT2 = x.T.astype(jnp.int32) * 2

    emb2 = jnp.concatenate(
        [emb_table, jnp.zeros_like(emb_table)], axis=1).reshape(2 * V, D)

    padded = _gather_padded(xT2, emb2, pos_table, B=B, S=S, D=D, NC=NC, NS=NS)
    return padded.reshape(B, S, 2 * D)[:, :, :D]
